# baseline (speedup 1.0000x reference)
import functools

import jax
import jax.numpy as jnp
from jax import lax
from jax.experimental import pallas as pl
from jax.experimental.pallas import tpu as pltpu
from jax.experimental.pallas import tpu_sc as plsc

B = 16384
D = 64
MARGIN = 1.0
PAD_IDX = 0

NC = 2
NS = 16
L = 16
NW = NC * NS
BPW = B // NW
IDX_CHUNK = 128
N_CHUNKS = BPW // IDX_CHUNK
GROUPS = BPW // L
DC = D // L


def _tl_kernel(x_hbm, tgt_hbm, nid_hbm, emb_hbm, out_hbm,
               tgt_v, nid_v, x_v, pos_v, neg_v, out_v, sem):
    wid = lax.axis_index("s") * NC + lax.axis_index("c")
    base = wid * BPW

    pltpu.sync_copy(tgt_hbm.at[pl.ds(base, BPW)], tgt_v)
    pltpu.sync_copy(nid_hbm.at[pl.ds(base, BPW)], nid_v)

    copies = []
    for j in range(N_CHUNKS):
        s = pl.ds(j * IDX_CHUNK, IDX_CHUNK)
        copies.append(pltpu.async_copy(emb_hbm.at[tgt_v.at[s]], pos_v.at[s], sem))
        copies.append(pltpu.async_copy(emb_hbm.at[nid_v.at[s]], neg_v.at[s], sem))
    pltpu.sync_copy(x_hbm.at[pl.ds(base, BPW)], x_v)
    for c in copies:
        c.wait()

    row_iota = lax.iota(jnp.int32, L)
    perms = [row_iota ^ k for k in (8, 4, 2, 1)]

    dnums = lax.GatherDimensionNumbers(
        offset_dims=(), collapsed_slice_dims=(0,), start_index_map=(0,))

    def lane_perm(v, idx):
        return lax.gather(v, idx[:, None], dnums, slice_sizes=(1,),
                          mode=lax.GatherScatterMode.PROMISE_IN_BOUNDS)

    def hsum(v):
        for p in perms:
            v = v + lane_perm(v, p)
        return v

    def group_body(g, carry):
        rb = g * L
        ovec = jnp.zeros((L,), jnp.float32)
        for i in range(L):
            r = rb + i
            acc = jnp.zeros((L,), jnp.float32)
            for c in range(DC):
                cs = pl.ds(c * L, L)
                xc = x_v[r, cs]
                pc = pos_v[r, cs]
                nc = neg_v[r, cs]
                acc = acc + (pc * pc - nc * nc - (xc + xc) * (pc - nc))
            ovec = jnp.where(row_iota == i, hsum(acc), ovec)
        tg = tgt_v[pl.ds(rb, L)]
        loss = jnp.maximum(ovec + MARGIN, 0.0)
        out_v[pl.ds(rb, L)] = jnp.where(tg == PAD_IDX, 0.0, loss)
        return carry

    lax.fori_loop(0, GROUPS, group_body, 0)

    pltpu.sync_copy(out_v, out_hbm.at[pl.ds(base, BPW)])


@functools.partial(
    pl.kernel,
    mesh=plsc.VectorSubcoreMesh(core_axis_name="c", subcore_axis_name="s"),
    compiler_params=pltpu.CompilerParams(use_tc_tiling_on_sc=False),
    out_type=jax.ShapeDtypeStruct((B,), jnp.float32),
    scratch_types=[
        pltpu.VMEM((BPW,), jnp.int32),
        pltpu.VMEM((BPW,), jnp.int32),
        pltpu.VMEM((BPW, D), jnp.float32),
        pltpu.VMEM((BPW, D), jnp.float32),
        pltpu.VMEM((BPW, D), jnp.float32),
        pltpu.VMEM((BPW,), jnp.float32),
        pltpu.SemaphoreType.DMA,
    ],
)
def _tl_call(x_hbm, tgt_hbm, nid_hbm, emb_hbm, out_hbm,
             tgt_v, nid_v, x_v, pos_v, neg_v, out_v, sem):
    _tl_kernel(x_hbm, tgt_hbm, nid_hbm, emb_hbm, out_hbm,
               tgt_v, nid_v, x_v, pos_v, neg_v, out_v, sem)


def kernel(x, targets, emb, neg_ids):
    return _tl_call(x, targets, neg_ids, emb)
